# baseline (speedup 1.0000x reference)
import functools

import jax
import jax.numpy as jnp
from jax import lax
from jax.experimental import pallas as pl
from jax.experimental.pallas import tpu as pltpu
from jax.experimental.pallas import tpu_sc as plsc

D = 128
CHUNK = 88
IGRP = 4
NBUF = 4
NWORKERS = 32
JUNK = 16


def _make_agg(npad, nchunk):
    mesh = plsc.VectorSubcoreMesh(core_axis_name="c", subcore_axis_name="s")
    rows_per_tile = npad // 16

    @functools.partial(
        pl.kernel,
        out_type=jax.ShapeDtypeStruct((2, npad, D), jnp.float32),
        mesh=mesh,
        scratch_types=[
            pltpu.VMEM((3, IGRP, CHUNK), jnp.int32),
            pltpu.VMEM((3, IGRP, CHUNK), jnp.int32),
            pltpu.VMEM((NBUF, CHUNK, D), jnp.float32),
            pltpu.VMEM_SHARED((npad, D), jnp.float32),
            pltpu.SemaphoreType.DMA((NBUF,)),
            pltpu.SemaphoreType.DMA((NBUF,)),
            pltpu.SemaphoreType.DMA((3,)),
            pltpu.SemaphoreType.DMA((3,)),
        ],
    )
    def agg(h_hbm, zeros_hbm, src_hbm, dst_hbm, out_hbm,
            src_v, dst_v, rows_v, acc_sh, sg, ss, sis, sid):
        ngrp = nchunk // IGRP
        c = lax.axis_index("c")
        s = lax.axis_index("s")
        w = s * 2 + c
        pltpu.sync_copy(zeros_hbm.at[pl.ds(s * rows_per_tile, rows_per_tile)],
                        acc_sh.at[pl.ds(s * rows_per_tile, rows_per_tile)])
        pltpu.sync_copy(src_hbm.at[w, pl.ds(0, IGRP)], src_v.at[0])
        pltpu.sync_copy(dst_hbm.at[w, pl.ds(0, IGRP)], dst_v.at[0])
        pltpu.sync_copy(src_hbm.at[w, pl.ds(IGRP, IGRP)], src_v.at[1])
        pltpu.sync_copy(dst_hbm.at[w, pl.ds(IGRP, IGRP)], dst_v.at[1])
        plsc.subcore_barrier()
        for jj in range(NBUF - 1):
            pltpu.async_copy(h_hbm.at[src_v.at[jj // IGRP, jj % IGRP]],
                             rows_v.at[jj], sg.at[jj])

        def body(j, carry):
            b = lax.rem(j, NBUF)
            g = lax.div(j, IGRP)
            k = lax.rem(j, IGRP)
            gs = lax.rem(g, 3)
            pltpu.make_async_copy(h_hbm.at[src_v.at[gs, k]], rows_v.at[b],
                                  sg.at[b]).wait()
            pltpu.async_copy(rows_v.at[b], acc_sh.at[dst_v.at[gs, k]],
                             ss.at[b], add=True)

            @pl.when(j >= 1)
            def _():
                jp = j - 1
                bp = lax.rem(jp, NBUF)
                pltpu.make_async_copy(
                    rows_v.at[bp],
                    acc_sh.at[dst_v.at[lax.rem(lax.div(jp, IGRP), 3),
                                       lax.rem(jp, IGRP)]],
                    ss.at[bp]).wait()

            m = j + NBUF - 1

            @pl.when(m < nchunk)
            def _():
                gm = lax.div(m, IGRP)
                gms = lax.rem(gm, 3)

                @pl.when((lax.rem(m, IGRP) == 0) & (gm >= 2))
                def _():
                    pltpu.make_async_copy(
                        src_hbm.at[w, pl.ds(gm * IGRP, IGRP)],
                        src_v.at[gms], sis.at[gms]).wait()
                    pltpu.make_async_copy(
                        dst_hbm.at[w, pl.ds(gm * IGRP, IGRP)],
                        dst_v.at[gms], sid.at[gms]).wait()

                @pl.when((lax.rem(m, IGRP) == 0) & (gm + 1 < ngrp))
                def _():
                    gn = gm + 1
                    gns = lax.rem(gn, 3)
                    pltpu.async_copy(src_hbm.at[w, pl.ds(gn * IGRP, IGRP)],
                                     src_v.at[gns], sis.at[gns])
                    pltpu.async_copy(dst_hbm.at[w, pl.ds(gn * IGRP, IGRP)],
                                     dst_v.at[gns], sid.at[gns])

                pltpu.async_copy(h_hbm.at[src_v.at[gms, lax.rem(m, IGRP)]],
                                 rows_v.at[lax.rem(m, NBUF)],
                                 sg.at[lax.rem(m, NBUF)])

            return carry

        lax.fori_loop(0, nchunk, body, 0)
        jl = nchunk - 1
        pltpu.make_async_copy(
            rows_v.at[jl % NBUF],
            acc_sh.at[dst_v.at[(jl // IGRP) % 3, jl % IGRP]],
            ss.at[jl % NBUF]).wait()
        plsc.subcore_barrier()
        pltpu.sync_copy(acc_sh.at[pl.ds(s * rows_per_tile, rows_per_tile)],
                        out_hbm.at[c, pl.ds(s * rows_per_tile, rows_per_tile)])

    return agg


def _matmul_body(h_ref, w_ref, b_ref, out_ref):
    out_ref[...] = (
        jnp.dot(h_ref[...], w_ref[...], preferred_element_type=jnp.float32)
        + b_ref[...]
    )


def _matmul(h, w, b, rb):
    npad = h.shape[0]
    grid = npad // rb
    return pl.pallas_call(
        _matmul_body,
        grid=(grid,),
        in_specs=[
            pl.BlockSpec((rb, D), lambda i: (i, 0)),
            pl.BlockSpec((D, D), lambda i: (0, 0)),
            pl.BlockSpec((1, D), lambda i: (0, 0)),
        ],
        out_specs=pl.BlockSpec((rb, D), lambda i: (i, 0)),
        out_shape=jax.ShapeDtypeStruct((npad, D), jnp.float32),
    )(h, w, b)


def _mlp_body(h_ref, agg_ref, wa_ref, ba_ref, wb_ref, bb_ref, out_ref):
    z = h_ref[...] + agg_ref[0] + agg_ref[1]
    z = jnp.maximum(
        jnp.dot(z, wa_ref[...], preferred_element_type=jnp.float32)
        + ba_ref[...],
        0.0,
    )
    out_ref[...] = (
        jnp.dot(z, wb_ref[...], preferred_element_type=jnp.float32)
        + bb_ref[...]
    )


def _mlp(h, agg, wa, ba, wb, bb, rb):
    npad = h.shape[0]
    grid = npad // rb
    return pl.pallas_call(
        _mlp_body,
        grid=(grid,),
        in_specs=[
            pl.BlockSpec((rb, D), lambda i: (i, 0)),
            pl.BlockSpec((2, rb, D), lambda i: (0, i, 0)),
            pl.BlockSpec((D, D), lambda i: (0, 0)),
            pl.BlockSpec((1, D), lambda i: (0, 0)),
            pl.BlockSpec((D, D), lambda i: (0, 0)),
            pl.BlockSpec((1, D), lambda i: (0, 0)),
        ],
        out_specs=pl.BlockSpec((rb, D), lambda i: (i, 0)),
        out_shape=jax.ShapeDtypeStruct((npad, D), jnp.float32),
    )(h, agg, wa, ba, wb, bb)


def kernel(h, x, edge_index, params):
    p = params
    n = h.shape[0]
    e = edge_index.shape[1]
    npad = ((n + JUNK + 127) // 128) * 128
    nchunk = -(-e // (NWORKERS * CHUNK))
    nchunk = ((nchunk + IGRP - 1) // IGRP) * IGRP
    epad = NWORKERS * nchunk * CHUNK
    rb = npad // 4

    pad = epad - e
    ar = jnp.arange(pad, dtype=jnp.int32)
    src_p = jnp.concatenate([edge_index[0], (ar * 37) % n])
    dst_p = jnp.concatenate([edge_index[1], n + ar % (npad - n)])
    src3 = src_p.reshape(NWORKERS, nchunk, CHUNK)
    dst3 = dst_p.reshape(NWORKERS, nchunk, CHUNK)
    zeros = jnp.zeros((npad, D), jnp.float32)

    hp = jnp.pad(h, ((0, npad - n), (0, 0)))
    agg_fn = _make_agg(npad, nchunk)

    hcur = _matmul(hp, p["W_emb"], p["b_emb"].reshape(1, D), rb)
    for i in range(6):
        agg = agg_fn(hcur, zeros, src3, dst3)
        hcur = _mlp(hcur, agg, p[f"W{i}a"], p[f"b{i}a"].reshape(1, D),
                    p[f"W{i}b"], p[f"b{i}b"].reshape(1, D), rb)
    hout = _matmul(hcur, p["W_out"], p["b_out"].reshape(1, D), rb)
    return (hout[:n], x)


# speedup vs baseline: 1.0410x; 1.0410x over previous
import functools

import jax
import jax.numpy as jnp
from jax import lax
from jax.experimental import pallas as pl
from jax.experimental.pallas import tpu as pltpu
from jax.experimental.pallas import tpu_sc as plsc

D = 128
CHUNK = 120
IGRP = 4
NBUF = 3
NWORKERS = 32
JUNK = 16


def _make_agg(npad, nchunk):
    mesh = plsc.VectorSubcoreMesh(core_axis_name="c", subcore_axis_name="s")
    rows_per_tile = npad // 16

    @functools.partial(
        pl.kernel,
        out_type=jax.ShapeDtypeStruct((2, npad, D), jnp.float32),
        mesh=mesh,
        scratch_types=[
            pltpu.VMEM((3, IGRP, CHUNK), jnp.int32),
            pltpu.VMEM((3, IGRP, CHUNK), jnp.int32),
            pltpu.VMEM((NBUF, CHUNK, D), jnp.float32),
            pltpu.VMEM_SHARED((npad, D), jnp.float32),
            pltpu.SemaphoreType.DMA((NBUF,)),
            pltpu.SemaphoreType.DMA((NBUF,)),
            pltpu.SemaphoreType.DMA((3,)),
            pltpu.SemaphoreType.DMA((3,)),
        ],
    )
    def agg(h_hbm, zeros_hbm, src_hbm, dst_hbm, out_hbm,
            src_v, dst_v, rows_v, acc_sh, sg, ss, sis, sid):
        ngrp = nchunk // IGRP
        c = lax.axis_index("c")
        s = lax.axis_index("s")
        w = s * 2 + c
        zero = pltpu.async_copy(
            zeros_hbm.at[pl.ds(s * rows_per_tile, rows_per_tile)],
            acc_sh.at[pl.ds(s * rows_per_tile, rows_per_tile)], sis.at[2])
        pltpu.sync_copy(src_hbm.at[w, pl.ds(0, IGRP)], src_v.at[0])
        pltpu.sync_copy(dst_hbm.at[w, pl.ds(0, IGRP)], dst_v.at[0])
        pltpu.sync_copy(src_hbm.at[w, pl.ds(IGRP, IGRP)], src_v.at[1])
        pltpu.sync_copy(dst_hbm.at[w, pl.ds(IGRP, IGRP)], dst_v.at[1])
        zero.wait()
        plsc.subcore_barrier()
        for jj in range(NBUF - 1):
            pltpu.async_copy(h_hbm.at[src_v.at[jj // IGRP, jj % IGRP]],
                             rows_v.at[jj], sg.at[jj])

        def body(j, carry):
            b = lax.rem(j, NBUF)
            g = lax.div(j, IGRP)
            k = lax.rem(j, IGRP)
            gs = lax.rem(g, 3)
            pltpu.make_async_copy(h_hbm.at[src_v.at[gs, k]], rows_v.at[b],
                                  sg.at[b]).wait()
            pltpu.async_copy(rows_v.at[b], acc_sh.at[dst_v.at[gs, k]],
                             ss.at[b], add=True)

            @pl.when(j >= 1)
            def _():
                jp = j - 1
                bp = lax.rem(jp, NBUF)
                pltpu.make_async_copy(
                    rows_v.at[bp],
                    acc_sh.at[dst_v.at[lax.rem(lax.div(jp, IGRP), 3),
                                       lax.rem(jp, IGRP)]],
                    ss.at[bp]).wait()

            m = j + NBUF - 1

            @pl.when(m < nchunk)
            def _():
                gm = lax.div(m, IGRP)
                gms = lax.rem(gm, 3)

                @pl.when((lax.rem(m, IGRP) == 0) & (gm >= 2))
                def _():
                    pltpu.make_async_copy(
                        src_hbm.at[w, pl.ds(gm * IGRP, IGRP)],
                        src_v.at[gms], sis.at[gms]).wait()
                    pltpu.make_async_copy(
                        dst_hbm.at[w, pl.ds(gm * IGRP, IGRP)],
                        dst_v.at[gms], sid.at[gms]).wait()

                @pl.when((lax.rem(m, IGRP) == 0) & (gm + 1 < ngrp))
                def _():
                    gn = gm + 1
                    gns = lax.rem(gn, 3)
                    pltpu.async_copy(src_hbm.at[w, pl.ds(gn * IGRP, IGRP)],
                                     src_v.at[gns], sis.at[gns])
                    pltpu.async_copy(dst_hbm.at[w, pl.ds(gn * IGRP, IGRP)],
                                     dst_v.at[gns], sid.at[gns])

                pltpu.async_copy(h_hbm.at[src_v.at[gms, lax.rem(m, IGRP)]],
                                 rows_v.at[lax.rem(m, NBUF)],
                                 sg.at[lax.rem(m, NBUF)])

            return carry

        lax.fori_loop(0, nchunk, body, 0)
        jl = nchunk - 1
        pltpu.make_async_copy(
            rows_v.at[jl % NBUF],
            acc_sh.at[dst_v.at[(jl // IGRP) % 3, jl % IGRP]],
            ss.at[jl % NBUF]).wait()
        plsc.subcore_barrier()
        pltpu.sync_copy(acc_sh.at[pl.ds(s * rows_per_tile, rows_per_tile)],
                        out_hbm.at[c, pl.ds(s * rows_per_tile, rows_per_tile)])

    return agg


def _matmul_body(h_ref, w_ref, b_ref, out_ref):
    out_ref[...] = (
        jnp.dot(h_ref[...], w_ref[...], preferred_element_type=jnp.float32)
        + b_ref[...]
    )


def _matmul(h, w, b, rb):
    npad = h.shape[0]
    grid = npad // rb
    return pl.pallas_call(
        _matmul_body,
        grid=(grid,),
        in_specs=[
            pl.BlockSpec((rb, D), lambda i: (i, 0)),
            pl.BlockSpec((D, D), lambda i: (0, 0)),
            pl.BlockSpec((1, D), lambda i: (0, 0)),
        ],
        out_specs=pl.BlockSpec((rb, D), lambda i: (i, 0)),
        out_shape=jax.ShapeDtypeStruct((npad, D), jnp.float32),
    )(h, w, b)


def _mlp_body(h_ref, agg_ref, wa_ref, ba_ref, wb_ref, bb_ref, out_ref):
    z = h_ref[...] + agg_ref[0] + agg_ref[1]
    z = jnp.maximum(
        jnp.dot(z, wa_ref[...], preferred_element_type=jnp.float32)
        + ba_ref[...],
        0.0,
    )
    out_ref[...] = (
        jnp.dot(z, wb_ref[...], preferred_element_type=jnp.float32)
        + bb_ref[...]
    )


def _mlp3_body(h_ref, agg_ref, wa_ref, ba_ref, wb_ref, bb_ref,
               wo_ref, bo_ref, out_ref):
    z = h_ref[...] + agg_ref[0] + agg_ref[1]
    z = jnp.maximum(
        jnp.dot(z, wa_ref[...], preferred_element_type=jnp.float32)
        + ba_ref[...],
        0.0,
    )
    z = (jnp.dot(z, wb_ref[...], preferred_element_type=jnp.float32)
         + bb_ref[...])
    out_ref[...] = (
        jnp.dot(z, wo_ref[...], preferred_element_type=jnp.float32)
        + bo_ref[...]
    )


def _mlp3(h, agg, wa, ba, wb, bb, wo, bo, rb):
    npad = h.shape[0]
    grid = npad // rb
    wspec = pl.BlockSpec((D, D), lambda i: (0, 0))
    bspec = pl.BlockSpec((1, D), lambda i: (0, 0))
    return pl.pallas_call(
        _mlp3_body,
        grid=(grid,),
        in_specs=[
            pl.BlockSpec((rb, D), lambda i: (i, 0)),
            pl.BlockSpec((2, rb, D), lambda i: (0, i, 0)),
            wspec, bspec, wspec, bspec, wspec, bspec,
        ],
        out_specs=pl.BlockSpec((rb, D), lambda i: (i, 0)),
        out_shape=jax.ShapeDtypeStruct((npad, D), jnp.float32),
    )(h, agg, wa, ba, wb, bb, wo, bo)


def _mlp(h, agg, wa, ba, wb, bb, rb):
    npad = h.shape[0]
    grid = npad // rb
    return pl.pallas_call(
        _mlp_body,
        grid=(grid,),
        in_specs=[
            pl.BlockSpec((rb, D), lambda i: (i, 0)),
            pl.BlockSpec((2, rb, D), lambda i: (0, i, 0)),
            pl.BlockSpec((D, D), lambda i: (0, 0)),
            pl.BlockSpec((1, D), lambda i: (0, 0)),
            pl.BlockSpec((D, D), lambda i: (0, 0)),
            pl.BlockSpec((1, D), lambda i: (0, 0)),
        ],
        out_specs=pl.BlockSpec((rb, D), lambda i: (i, 0)),
        out_shape=jax.ShapeDtypeStruct((npad, D), jnp.float32),
    )(h, agg, wa, ba, wb, bb)


def kernel(h, x, edge_index, params):
    p = params
    n = h.shape[0]
    e = edge_index.shape[1]
    npad = ((n + JUNK + 127) // 128) * 128
    nchunk = -(-e // (NWORKERS * CHUNK))
    nchunk = ((nchunk + IGRP - 1) // IGRP) * IGRP
    epad = NWORKERS * nchunk * CHUNK
    rb = npad // 4

    pad = epad - e
    ar = jnp.arange(pad, dtype=jnp.int32)
    src_p = jnp.concatenate([edge_index[0], (ar * 37) % n])
    dst_p = jnp.concatenate([edge_index[1], n + ar % (npad - n)])
    src3 = src_p.reshape(NWORKERS, nchunk, CHUNK)
    dst3 = dst_p.reshape(NWORKERS, nchunk, CHUNK)
    zeros = jnp.zeros((npad, D), jnp.float32)

    hp = jnp.pad(h, ((0, npad - n), (0, 0)))
    agg_fn = _make_agg(npad, nchunk)

    hcur = _matmul(hp, p["W_emb"], p["b_emb"].reshape(1, D), rb)
    for i in range(5):
        agg = agg_fn(hcur, zeros, src3, dst3)
        hcur = _mlp(hcur, agg, p[f"W{i}a"], p[f"b{i}a"].reshape(1, D),
                    p[f"W{i}b"], p[f"b{i}b"].reshape(1, D), rb)
    agg = agg_fn(hcur, zeros, src3, dst3)
    hout = _mlp3(hcur, agg, p["W5a"], p["b5a"].reshape(1, D),
                 p["W5b"], p["b5b"].reshape(1, D),
                 p["W_out"], p["b_out"].reshape(1, D), rb)
    return (hout[:n], x)


# speedup vs baseline: 1.0461x; 1.0049x over previous
import functools

import jax
import jax.numpy as jnp
from jax import lax
from jax.experimental import pallas as pl
from jax.experimental.pallas import tpu as pltpu
from jax.experimental.pallas import tpu_sc as plsc

D = 128
CHUNK = 120
IGRP = 4
NBUF = 3
NWORKERS = 32
JUNK = 16


def _make_agg(npad, nchunk):
    mesh = plsc.VectorSubcoreMesh(core_axis_name="c", subcore_axis_name="s")
    rows_per_tile = npad // 16

    @functools.partial(
        pl.kernel,
        out_type=jax.ShapeDtypeStruct((2, npad, D), jnp.float32),
        mesh=mesh,
        scratch_types=[
            pltpu.VMEM((3, IGRP, CHUNK), jnp.int32),
            pltpu.VMEM((3, IGRP, CHUNK), jnp.int32),
            pltpu.VMEM((NBUF, CHUNK, D), jnp.float32),
            pltpu.VMEM_SHARED((npad, D), jnp.float32),
            pltpu.SemaphoreType.DMA((NBUF,)),
            pltpu.SemaphoreType.DMA((NBUF,)),
            pltpu.SemaphoreType.DMA((3,)),
            pltpu.SemaphoreType.DMA((3,)),
        ],
    )
    def agg(h_hbm, zeros_hbm, src_hbm, dst_hbm, out_hbm,
            src_v, dst_v, rows_v, acc_sh, sg, ss, sis, sid):
        ngrp = nchunk // IGRP
        c = lax.axis_index("c")
        s = lax.axis_index("s")
        w = s * 2 + c
        zero = pltpu.async_copy(
            zeros_hbm.at[pl.ds(s * rows_per_tile, rows_per_tile)],
            acc_sh.at[pl.ds(s * rows_per_tile, rows_per_tile)], sis.at[2])
        pltpu.sync_copy(src_hbm.at[w, pl.ds(0, IGRP)], src_v.at[0])
        pltpu.sync_copy(dst_hbm.at[w, pl.ds(0, IGRP)], dst_v.at[0])
        pltpu.sync_copy(src_hbm.at[w, pl.ds(IGRP, IGRP)], src_v.at[1])
        pltpu.sync_copy(dst_hbm.at[w, pl.ds(IGRP, IGRP)], dst_v.at[1])
        zero.wait()
        plsc.subcore_barrier()
        for jj in range(NBUF - 1):
            pltpu.async_copy(h_hbm.at[src_v.at[jj // IGRP, jj % IGRP]],
                             rows_v.at[jj], sg.at[jj])

        def body(j, carry):
            b = lax.rem(j, NBUF)
            g = lax.div(j, IGRP)
            k = lax.rem(j, IGRP)
            gs = lax.rem(g, 3)
            pltpu.make_async_copy(h_hbm.at[src_v.at[gs, k]], rows_v.at[b],
                                  sg.at[b]).wait()
            pltpu.async_copy(rows_v.at[b], acc_sh.at[dst_v.at[gs, k]],
                             ss.at[b], add=True)

            @pl.when(j >= 1)
            def _():
                jp = j - 1
                bp = lax.rem(jp, NBUF)
                pltpu.make_async_copy(
                    rows_v.at[bp],
                    acc_sh.at[dst_v.at[lax.rem(lax.div(jp, IGRP), 3),
                                       lax.rem(jp, IGRP)]],
                    ss.at[bp]).wait()

            m = j + NBUF - 1

            @pl.when(m < nchunk)
            def _():
                gm = lax.div(m, IGRP)
                gms = lax.rem(gm, 3)

                @pl.when((lax.rem(m, IGRP) == 0) & (gm >= 2))
                def _():
                    pltpu.make_async_copy(
                        src_hbm.at[w, pl.ds(gm * IGRP, IGRP)],
                        src_v.at[gms], sis.at[gms]).wait()
                    pltpu.make_async_copy(
                        dst_hbm.at[w, pl.ds(gm * IGRP, IGRP)],
                        dst_v.at[gms], sid.at[gms]).wait()

                @pl.when((lax.rem(m, IGRP) == 0) & (gm + 1 < ngrp))
                def _():
                    gn = gm + 1
                    gns = lax.rem(gn, 3)
                    pltpu.async_copy(src_hbm.at[w, pl.ds(gn * IGRP, IGRP)],
                                     src_v.at[gns], sis.at[gns])
                    pltpu.async_copy(dst_hbm.at[w, pl.ds(gn * IGRP, IGRP)],
                                     dst_v.at[gns], sid.at[gns])

                pltpu.async_copy(h_hbm.at[src_v.at[gms, lax.rem(m, IGRP)]],
                                 rows_v.at[lax.rem(m, NBUF)],
                                 sg.at[lax.rem(m, NBUF)])

            return carry

        lax.fori_loop(0, nchunk, body, 0)
        jl = nchunk - 1
        pltpu.make_async_copy(
            rows_v.at[jl % NBUF],
            acc_sh.at[dst_v.at[(jl // IGRP) % 3, jl % IGRP]],
            ss.at[jl % NBUF]).wait()
        plsc.subcore_barrier()
        pltpu.sync_copy(acc_sh.at[pl.ds(s * rows_per_tile, rows_per_tile)],
                        out_hbm.at[c, pl.ds(s * rows_per_tile, rows_per_tile)])

    return agg


def _matmul_body(h_ref, w_ref, b_ref, out_ref):
    out_ref[...] = (
        jnp.dot(h_ref[...], w_ref[...], preferred_element_type=jnp.float32)
        + b_ref[...]
    )


def _matmul(h, w, b, rb):
    npad = h.shape[0]
    grid = npad // rb
    return pl.pallas_call(
        _matmul_body,
        grid=(grid,),
        in_specs=[
            pl.BlockSpec((rb, D), lambda i: (i, 0)),
            pl.BlockSpec((D, D), lambda i: (0, 0)),
            pl.BlockSpec((1, D), lambda i: (0, 0)),
        ],
        out_specs=pl.BlockSpec((rb, D), lambda i: (i, 0)),
        out_shape=jax.ShapeDtypeStruct((npad, D), jnp.float32),
    )(h, w, b)


def _mlp_body(h_ref, agg_ref, wa_ref, ba_ref, wb_ref, bb_ref, out_ref):
    z = h_ref[...] + agg_ref[0] + agg_ref[1]
    z = jnp.maximum(
        jnp.dot(z, wa_ref[...], preferred_element_type=jnp.float32)
        + ba_ref[...],
        0.0,
    )
    out_ref[...] = (
        jnp.dot(z, wb_ref[...], preferred_element_type=jnp.float32)
        + bb_ref[...]
    )


def _mlp3_body(h_ref, agg_ref, wa_ref, ba_ref, wb_ref, bb_ref,
               wo_ref, bo_ref, out_ref):
    z = h_ref[...] + agg_ref[0] + agg_ref[1]
    z = jnp.maximum(
        jnp.dot(z, wa_ref[...], preferred_element_type=jnp.float32)
        + ba_ref[...],
        0.0,
    )
    z = (jnp.dot(z, wb_ref[...], preferred_element_type=jnp.float32)
         + bb_ref[...])
    out_ref[...] = (
        jnp.dot(z, wo_ref[...], preferred_element_type=jnp.float32)
        + bo_ref[...]
    )


def _mlp3(h, agg, wa, ba, wb, bb, wo, bo, rb):
    npad = h.shape[0]
    grid = npad // rb
    wspec = pl.BlockSpec((D, D), lambda i: (0, 0))
    bspec = pl.BlockSpec((1, D), lambda i: (0, 0))
    return pl.pallas_call(
        _mlp3_body,
        grid=(grid,),
        in_specs=[
            pl.BlockSpec((rb, D), lambda i: (i, 0)),
            pl.BlockSpec((2, rb, D), lambda i: (0, i, 0)),
            wspec, bspec, wspec, bspec, wspec, bspec,
        ],
        out_specs=pl.BlockSpec((rb, D), lambda i: (i, 0)),
        out_shape=jax.ShapeDtypeStruct((npad, D), jnp.float32),
    )(h, agg, wa, ba, wb, bb, wo, bo)


def _mlp(h, agg, wa, ba, wb, bb, rb):
    npad = h.shape[0]
    grid = npad // rb
    return pl.pallas_call(
        _mlp_body,
        grid=(grid,),
        in_specs=[
            pl.BlockSpec((rb, D), lambda i: (i, 0)),
            pl.BlockSpec((2, rb, D), lambda i: (0, i, 0)),
            pl.BlockSpec((D, D), lambda i: (0, 0)),
            pl.BlockSpec((1, D), lambda i: (0, 0)),
            pl.BlockSpec((D, D), lambda i: (0, 0)),
            pl.BlockSpec((1, D), lambda i: (0, 0)),
        ],
        out_specs=pl.BlockSpec((rb, D), lambda i: (i, 0)),
        out_shape=jax.ShapeDtypeStruct((npad, D), jnp.float32),
    )(h, agg, wa, ba, wb, bb)


def kernel(h, x, edge_index, params):
    p = params
    n = h.shape[0]
    e = edge_index.shape[1]
    npad = ((n + JUNK + 127) // 128) * 128
    nchunk = -(-e // (NWORKERS * CHUNK))
    nchunk = ((nchunk + IGRP - 1) // IGRP) * IGRP
    epad = NWORKERS * nchunk * CHUNK
    rb = npad // 2

    pad = epad - e
    ar = jnp.arange(pad, dtype=jnp.int32)
    src_p = jnp.concatenate([edge_index[0], (ar * 37) % n])
    dst_p = jnp.concatenate([edge_index[1], n + ar % (npad - n)])
    src3 = src_p.reshape(NWORKERS, nchunk, CHUNK)
    dst3 = dst_p.reshape(NWORKERS, nchunk, CHUNK)
    zeros = jnp.zeros((npad, D), jnp.float32)

    hp = jnp.pad(h, ((0, npad - n), (0, 0)))
    agg_fn = _make_agg(npad, nchunk)

    hcur = _matmul(hp, p["W_emb"], p["b_emb"].reshape(1, D), rb)
    for i in range(5):
        agg = agg_fn(hcur, zeros, src3, dst3)
        hcur = _mlp(hcur, agg, p[f"W{i}a"], p[f"b{i}a"].reshape(1, D),
                    p[f"W{i}b"], p[f"b{i}b"].reshape(1, D), rb)
    agg = agg_fn(hcur, zeros, src3, dst3)
    hout = _mlp3(hcur, agg, p["W5a"], p["b5a"].reshape(1, D),
                 p["W5b"], p["b5b"].reshape(1, D),
                 p["W_out"], p["b_out"].reshape(1, D), rb)
    return (hout[:n], x)
